# baseline (speedup 1.0000x reference)
import functools

import jax
import jax.numpy as jnp
from jax import lax
from jax.experimental import pallas as pl
from jax.experimental.pallas import tpu as pltpu
from jax.experimental.pallas import tpu_sc as plsc

_INFO = plsc.get_sparse_core_info()
_NC, _NS = _INFO.num_cores, _INFO.num_subcores
_NW = _NC * _NS

_D = 16
_C = 1024
_BATCH = 16384
_HIST = 200
_GRP = _BATCH // _C
_NCHUNK = _HIST * _GRP
_PER_W = _NCHUNK // _NW


def _build():
    mesh = plsc.VectorSubcoreMesh(core_axis_name="c", subcore_axis_name="s")
    n_out = _HIST * 2 * (_BATCH // 128) * 8 * 128

    @functools.partial(
        pl.kernel,
        out_type=jax.ShapeDtypeStruct((n_out,), jnp.float32),
        mesh=mesh,
        scratch_types=[
            pltpu.VMEM((_C,), jnp.int32),
            pltpu.VMEM((_C,), jnp.int32),
            pltpu.VMEM((_C, _D), jnp.float32),
            pltpu.VMEM((_C, _D), jnp.float32),
            pltpu.VMEM((2 * 8192,), jnp.float32),
            pltpu.VMEM((2 * 8192,), jnp.float32),
            pltpu.SemaphoreType.DMA,
            pltpu.SemaphoreType.DMA,
            pltpu.SemaphoreType.DMA,
            pltpu.SemaphoreType.DMA,
            pltpu.SemaphoreType.DMA,
            pltpu.SemaphoreType.DMA,
        ],
        compiler_params=pltpu.CompilerParams(use_tc_tiling_on_sc=False,
                                             needs_layout_passes=False),
    )
    def body(flat_hbm, table_hbm, out_hbm, idx0, idx1, rows0, rows1, tb0, tb1,
             si0, si1, sg0, sg1, so0, so1):
        wid = lax.axis_index("s") * _NC + lax.axis_index("c")
        c_base = wid * _PER_W
        idxs = (idx0, idx1)
        rows = (rows0, rows1)
        tbs = (tb0, tb1)
        si = (si0, si1)
        sg = (sg0, sg1)
        so = (so0, so1)
        iota = lax.iota(jnp.int32, 16)
        k_vec = ((iota >> 3) << 13) + ((iota & 7) << 7)

        def idx_start(c, b):
            pltpu.async_copy(flat_hbm.at[pl.ds((c_base + c) * _C, _C)],
                             idxs[b], si[b])

        def idx_wait(b):
            pltpu.make_async_copy(flat_hbm.at[pl.ds(0, _C)], idxs[b],
                                  si[b]).wait()

        def g_start(b):
            pltpu.async_copy(table_hbm.at[idxs[b]], rows[b], sg[b])

        def g_wait(b):
            pltpu.make_async_copy(table_hbm.at[idxs[b]], rows[b],
                                  sg[b]).wait()

        def out_start(c, b):
            cg = c_base + c
            h = cg // _GRP
            btg = cg - h * _GRP
            for dt in range(2):
                e0 = h * 262144 + dt * 131072 + btg * 8192
                pltpu.async_copy(tbs[b].at[pl.ds(dt * 8192, 8192)],
                                 out_hbm.at[pl.ds(e0, 8192)], so[b])

        def out_wait(b):
            for dt in range(2):
                pltpu.make_async_copy(tbs[b].at[pl.ds(dt * 8192, 8192)],
                                      out_hbm.at[pl.ds(0, 8192)],
                                      so[b]).wait()

        def transpose(b):
            rb = rows[b]
            tb = tbs[b]

            def tloop(bt, carry):
                sbase = bt * 1024
                jbase = bt * 128
                for l in range(128):
                    v = rb[jbase + l]
                    plsc.store_scatter(tb, [k_vec + (sbase + l)], v)
                return carry

            lax.fori_loop(0, 8, tloop, 0)

        idx_start(0, 0)
        idx_start(1, 1)
        idx_wait(0)
        g_start(0)

        g_wait(0)
        idx_start(2, 0)
        idx_wait(1)
        g_start(1)
        transpose(0)
        out_start(0, 0)

        g_wait(1)
        idx_start(3, 1)
        idx_wait(0)
        g_start(0)
        transpose(1)
        out_start(1, 1)

        def pair(g, carry):
            c0 = 2 * g
            g_wait(0)
            idx_start(c0 + 2, 0)
            idx_wait(1)
            g_start(1)
            out_wait(0)
            transpose(0)
            out_start(c0, 0)

            g_wait(1)
            idx_start(c0 + 3, 1)
            idx_wait(0)
            g_start(0)
            out_wait(1)
            transpose(1)
            out_start(c0 + 1, 1)
            return carry

        lax.fori_loop(1, _PER_W // 2 - 1, pair, 0)

        g_wait(0)
        idx_wait(1)
        g_start(1)
        out_wait(0)
        transpose(0)
        out_start(_PER_W - 2, 0)

        g_wait(1)
        out_wait(1)
        transpose(1)
        out_start(_PER_W - 1, 1)

        out_wait(0)
        out_wait(1)

    return body


_LOOKUP = _build()


def kernel(sentence, table):
    b, h = sentence.shape
    d = table.shape[1]
    flat_t = sentence.T.reshape(-1).astype(jnp.int32)
    out2 = _LOOKUP(flat_t, table)
    out = out2.reshape(h, 2, b // 128, 8, 128).transpose(2, 4, 0, 1, 3)
    return out.reshape(b, h, d)


# speedup vs baseline: 1.7704x; 1.7704x over previous
import functools

import jax
import jax.numpy as jnp
from jax import lax
from jax.experimental import pallas as pl
from jax.experimental.pallas import tpu as pltpu
from jax.experimental.pallas import tpu_sc as plsc

_INFO = plsc.get_sparse_core_info()
_NC, _NS = _INFO.num_cores, _INFO.num_subcores
_NW = _NC * _NS

_D = 16
_C = 1024
_BATCH = 16384
_HIST = 200
_GRP = _BATCH // _C
_NCHUNK = _HIST * _GRP
_PER_W = _NCHUNK // _NW


def _build():
    mesh = plsc.VectorSubcoreMesh(core_axis_name="c", subcore_axis_name="s")
    n_out = _HIST * 2 * (_BATCH // 128) * 8 * 128

    @functools.partial(
        pl.kernel,
        out_type=jax.ShapeDtypeStruct((n_out,), jnp.float32),
        mesh=mesh,
        scratch_types=[
            pltpu.VMEM((_C,), jnp.int32),
            pltpu.VMEM((_C,), jnp.int32),
            pltpu.VMEM((_C, _D), jnp.float32),
            pltpu.VMEM((_C, _D), jnp.float32),
            pltpu.VMEM((2 * 8192,), jnp.float32),
            pltpu.VMEM((2 * 8192,), jnp.float32),
            pltpu.SemaphoreType.DMA,
            pltpu.SemaphoreType.DMA,
            pltpu.SemaphoreType.DMA,
            pltpu.SemaphoreType.DMA,
            pltpu.SemaphoreType.DMA,
            pltpu.SemaphoreType.DMA,
        ],
        compiler_params=pltpu.CompilerParams(use_tc_tiling_on_sc=False,
                                             needs_layout_passes=False),
    )
    def body(flat_hbm, table_hbm, out_hbm, idx0, idx1, rows0, rows1, tb0, tb1,
             si0, si1, sg0, sg1, so0, so1):
        wid = lax.axis_index("s") * _NC + lax.axis_index("c")
        c_base = wid * _PER_W
        idxs = (idx0, idx1)
        rows = (rows0, rows1)
        tbs = (tb0, tb1)
        si = (si0, si1)
        sg = (sg0, sg1)
        so = (so0, so1)
        iota = lax.iota(jnp.int32, 16)
        xs = [(d0 + iota) & 15 for d0 in range(16)]
        stb = [((x >> 3) << 13) + ((x & 7) << 7) + iota for x in xs]

        def idx_start(c, b):
            pltpu.async_copy(flat_hbm.at[pl.ds((c_base + c) * _C, _C)],
                             idxs[b], si[b])

        def idx_wait(b):
            pltpu.make_async_copy(flat_hbm.at[pl.ds(0, _C)], idxs[b],
                                  si[b]).wait()

        def g_start(b):
            pltpu.async_copy(table_hbm.at[idxs[b]], rows[b], sg[b])

        def g_wait(b):
            pltpu.make_async_copy(table_hbm.at[idxs[b]], rows[b],
                                  sg[b]).wait()

        def out_start(c, b):
            cg = c_base + c
            h = cg // _GRP
            btg = cg - h * _GRP
            for dt in range(2):
                e0 = h * 262144 + dt * 131072 + btg * 8192
                pltpu.async_copy(tbs[b].at[pl.ds(dt * 8192, 8192)],
                                 out_hbm.at[pl.ds(e0, 8192)], so[b])

        def out_wait(b):
            for dt in range(2):
                pltpu.make_async_copy(tbs[b].at[pl.ds(dt * 8192, 8192)],
                                      out_hbm.at[pl.ds(0, 8192)],
                                      so[b]).wait()

        def transpose(b):
            rb = rows[b]
            tb = tbs[b]

            def tloop(bt, carry):
                sb = bt * 128
                sb8 = bt * 1024
                for l0 in range(0, 128, 16):
                    tl = sb + l0 + iota
                    for d0 in range(16):
                        v = plsc.load_gather(rb, [tl, xs[d0]])
                        plsc.store_scatter(tb, [stb[d0] + (sb8 + l0)], v)
                return carry

            lax.fori_loop(0, 8, tloop, 0)

        idx_start(0, 0)
        idx_start(1, 1)
        idx_wait(0)
        g_start(0)

        g_wait(0)
        idx_start(2, 0)
        idx_wait(1)
        g_start(1)
        transpose(0)
        out_start(0, 0)

        g_wait(1)
        idx_start(3, 1)
        idx_wait(0)
        g_start(0)
        transpose(1)
        out_start(1, 1)

        def pair(g, carry):
            c0 = 2 * g
            g_wait(0)
            idx_start(c0 + 2, 0)
            idx_wait(1)
            g_start(1)
            out_wait(0)
            transpose(0)
            out_start(c0, 0)

            g_wait(1)
            idx_start(c0 + 3, 1)
            idx_wait(0)
            g_start(0)
            out_wait(1)
            transpose(1)
            out_start(c0 + 1, 1)
            return carry

        lax.fori_loop(1, _PER_W // 2 - 1, pair, 0)

        g_wait(0)
        idx_wait(1)
        g_start(1)
        out_wait(0)
        transpose(0)
        out_start(_PER_W - 2, 0)

        g_wait(1)
        out_wait(1)
        transpose(1)
        out_start(_PER_W - 1, 1)

        out_wait(0)
        out_wait(1)

    return body


_LOOKUP = _build()


def kernel(sentence, table):
    b, h = sentence.shape
    d = table.shape[1]
    flat_t = sentence.T.reshape(-1).astype(jnp.int32)
    out2 = _LOOKUP(flat_t, table)
    out = out2.reshape(h, 2, b // 128, 8, 128).transpose(2, 4, 0, 1, 3)
    return out.reshape(b, h, d)


# speedup vs baseline: 1.7706x; 1.0001x over previous
import functools

import jax
import jax.numpy as jnp
from jax import lax
from jax.experimental import pallas as pl
from jax.experimental.pallas import tpu as pltpu
from jax.experimental.pallas import tpu_sc as plsc

_INFO = plsc.get_sparse_core_info()
_NC, _NS = _INFO.num_cores, _INFO.num_subcores
_NW = _NC * _NS

_D = 16
_C = 1024
_BATCH = 16384
_HIST = 200
_GRP = _BATCH // _C
_NCHUNK = _HIST * _GRP
_PER_W = _NCHUNK // _NW


def _build():
    mesh = plsc.VectorSubcoreMesh(core_axis_name="c", subcore_axis_name="s")
    n_out = _HIST * 2 * (_BATCH // 128) * 8 * 128

    @functools.partial(
        pl.kernel,
        out_type=jax.ShapeDtypeStruct((n_out,), jnp.float32),
        mesh=mesh,
        scratch_types=[
            pltpu.VMEM((8, 128), jnp.int32),
            pltpu.VMEM((8, 128), jnp.int32),
            pltpu.VMEM((_C,), jnp.int32),
            pltpu.VMEM((_C,), jnp.int32),
            pltpu.VMEM((_C, _D), jnp.float32),
            pltpu.VMEM((_C, _D), jnp.float32),
            pltpu.VMEM((2 * 8192,), jnp.float32),
            pltpu.VMEM((2 * 8192,), jnp.float32),
            pltpu.SemaphoreType.DMA,
            pltpu.SemaphoreType.DMA,
            pltpu.SemaphoreType.DMA,
            pltpu.SemaphoreType.DMA,
            pltpu.SemaphoreType.DMA,
            pltpu.SemaphoreType.DMA,
        ],
        compiler_params=pltpu.CompilerParams(use_tc_tiling_on_sc=False,
                                             needs_layout_passes=False),
    )
    def body(s4_hbm, table_hbm, out_hbm, idx0, idx1, ix0, ix1, rows0, rows1,
             tb0, tb1, si0, si1, sg0, sg1, so0, so1):
        wid = lax.axis_index("s") * _NC + lax.axis_index("c")
        c_base = wid * _PER_W
        idxs = (idx0, idx1)
        ixs = (ix0, ix1)
        rows = (rows0, rows1)
        tbs = (tb0, tb1)
        si = (si0, si1)
        sg = (sg0, sg1)
        so = (so0, so1)
        iota = lax.iota(jnp.int32, 16)
        xs = [(d0 + iota) & 15 for d0 in range(16)]
        stb = [((x >> 3) << 13) + ((x & 7) << 7) + iota for x in xs]

        def idx_start(c, b):
            cg = c_base + c
            h = cg // _GRP
            btg = cg - h * _GRP
            pltpu.async_copy(
                s4_hbm.at[h >> 3, pl.ds(btg * 8, 8), h & 7, :],
                idxs[b], si[b])

        def idx_wait(b):
            pltpu.make_async_copy(s4_hbm.at[0, pl.ds(0, 8), 0, :], idxs[b],
                                  si[b]).wait()

        def repack(b):
            src2 = idxs[b]
            dst1 = ixs[b]
            for r in range(8):
                for l0 in range(0, 128, 16):
                    dst1[pl.ds(r * 128 + l0, 16)] = src2[r, pl.ds(l0, 16)]

        def g_start(b):
            pltpu.async_copy(table_hbm.at[ixs[b]], rows[b], sg[b])

        def g_wait(b):
            pltpu.make_async_copy(table_hbm.at[ixs[b]], rows[b],
                                  sg[b]).wait()

        def out_start(c, b):
            cg = c_base + c
            h = cg // _GRP
            btg = cg - h * _GRP
            for dt in range(2):
                e0 = h * 262144 + dt * 131072 + btg * 8192
                pltpu.async_copy(tbs[b].at[pl.ds(dt * 8192, 8192)],
                                 out_hbm.at[pl.ds(e0, 8192)], so[b])

        def out_wait(b):
            for dt in range(2):
                pltpu.make_async_copy(tbs[b].at[pl.ds(dt * 8192, 8192)],
                                      out_hbm.at[pl.ds(0, 8192)],
                                      so[b]).wait()

        def transpose(b):
            rb = rows[b]
            tb = tbs[b]

            def tloop(bt, carry):
                sb8 = bt * 1024
                sb = bt * 128
                for l0 in range(0, 128, 16):
                    tl = sb + l0 + iota
                    for d0 in range(16):
                        v = plsc.load_gather(rb, [tl, xs[d0]])
                        plsc.store_scatter(tb, [stb[d0] + (sb8 + l0)], v)
                return carry

            lax.fori_loop(0, 8, tloop, 0)

        idx_start(0, 0)
        idx_start(1, 1)
        idx_wait(0)
        repack(0)
        g_start(0)

        g_wait(0)
        idx_start(2, 0)
        idx_wait(1)
        repack(1)
        g_start(1)
        transpose(0)
        out_start(0, 0)

        g_wait(1)
        idx_start(3, 1)
        idx_wait(0)
        repack(0)
        g_start(0)
        transpose(1)
        out_start(1, 1)

        def pair(g, carry):
            c0 = 2 * g
            g_wait(0)
            idx_start(c0 + 2, 0)
            idx_wait(1)
            repack(1)
            g_start(1)
            out_wait(0)
            transpose(0)
            out_start(c0, 0)

            g_wait(1)
            idx_start(c0 + 3, 1)
            idx_wait(0)
            repack(0)
            g_start(0)
            out_wait(1)
            transpose(1)
            out_start(c0 + 1, 1)
            return carry

        lax.fori_loop(1, _PER_W // 2 - 1, pair, 0)

        g_wait(0)
        idx_wait(1)
        repack(1)
        g_start(1)
        out_wait(0)
        transpose(0)
        out_start(_PER_W - 2, 0)

        g_wait(1)
        out_wait(1)
        transpose(1)
        out_start(_PER_W - 1, 1)

        out_wait(0)
        out_wait(1)

    return body


_LOOKUP = _build()


def kernel(sentence, table):
    b, h = sentence.shape
    d = table.shape[1]
    s4 = (sentence.astype(jnp.int32).T.reshape(h // 8, 8, b // 128, 128)
          .transpose(0, 2, 1, 3))
    out2 = _LOOKUP(s4, table)
    out = out2.reshape(h, 2, b // 128, 8, 128).transpose(2, 4, 0, 1, 3)
    return out.reshape(b, h, d)
